# Initial kernel scaffold; baseline (speedup 1.0000x reference)
import jax
import jax.numpy as jnp
from jax import lax
from jax.experimental import pallas as pl
from jax.experimental.pallas import tpu as pltpu
from jax.experimental.pallas import tpu_sc as plsc

N = 10000
E = 320000
IN_DIM = 128
HID = 16
OUT_DIM = 7
NPAD = 10240
NC = 2
NS = 16
NW = NC * NS
EPT = E // NW
CHUNK = 2000
NCH = EPT // CHUNK
RPT = NPAD // NS

_mesh = plsc.VectorSubcoreMesh(
    core_axis_name="c", subcore_axis_name="s", num_cores=NC, num_subcores=NS
)


def _deg_body(dst_hbm, out_hbm, idx_v, ones_v, stg, deg_sh):
    c = lax.axis_index("c")
    s = lax.axis_index("s")
    wid = c * NS + s

    def fill(i, _):
        ones_v[pl.ds(i * 16, 16)] = jnp.full((16,), 1.0, jnp.float32)
        stg[pl.ds(i * 16, 16)] = jnp.zeros((16,), jnp.float32)
        return 0

    lax.fori_loop(0, CHUNK // 16, fill, 0)
    pltpu.sync_copy(stg.at[pl.ds(0, RPT)], deg_sh.at[pl.ds(s * RPT, RPT)])
    plsc.subcore_barrier()

    def body(j, _):
        base = wid * EPT + j * CHUNK
        pltpu.sync_copy(dst_hbm.at[pl.ds(base, CHUNK)], idx_v)
        pltpu.sync_copy(ones_v, deg_sh.at[idx_v], add=True)
        return 0

    lax.fori_loop(0, NCH, body, 0)
    plsc.subcore_barrier()
    pltpu.sync_copy(deg_sh.at[pl.ds(s * RPT, RPT)], stg.at[pl.ds(0, RPT)])
    pltpu.sync_copy(stg.at[pl.ds(0, RPT)], out_hbm.at[pl.ds(c * NPAD + s * RPT, RPT)])


_deg_call = pl.kernel(
    _deg_body,
    out_type=jax.ShapeDtypeStruct((2 * NPAD,), jnp.float32),
    mesh=_mesh,
    scratch_types=[
        pltpu.VMEM((CHUNK,), jnp.int32),
        pltpu.VMEM((CHUNK,), jnp.float32),
        pltpu.VMEM((CHUNK,), jnp.float32),
        pltpu.VMEM_SHARED((NPAD,), jnp.float32),
    ],
)


def _msg_body(tab_hbm, src_hbm, dst_hbm, out_hbm, idx_s, idx_d, rows, stg, acc_sh):
    c = lax.axis_index("c")
    s = lax.axis_index("s")
    wid = c * NS + s

    def zfill(i, _):
        stg[i, :] = jnp.zeros((16,), jnp.float32)
        return 0

    lax.fori_loop(0, RPT, zfill, 0)
    pltpu.sync_copy(stg, acc_sh.at[pl.ds(s * RPT, RPT)])
    plsc.subcore_barrier()

    def body(j, _):
        base = wid * EPT + j * CHUNK
        pltpu.sync_copy(src_hbm.at[pl.ds(base, CHUNK)], idx_s)
        pltpu.sync_copy(dst_hbm.at[pl.ds(base, CHUNK)], idx_d)
        pltpu.sync_copy(tab_hbm.at[idx_s], rows)
        pltpu.sync_copy(rows, acc_sh.at[idx_d], add=True)
        return 0

    lax.fori_loop(0, NCH, body, 0)
    plsc.subcore_barrier()
    pltpu.sync_copy(acc_sh.at[pl.ds(s * RPT, RPT)], stg)
    pltpu.sync_copy(stg, out_hbm.at[pl.ds(c * NPAD + s * RPT, RPT)])


_msg_call = pl.kernel(
    _msg_body,
    out_type=jax.ShapeDtypeStruct((2 * NPAD, HID), jnp.float32),
    mesh=_mesh,
    scratch_types=[
        pltpu.VMEM((CHUNK,), jnp.int32),
        pltpu.VMEM((CHUNK,), jnp.int32),
        pltpu.VMEM((CHUNK, HID), jnp.float32),
        pltpu.VMEM((RPT, HID), jnp.float32),
        pltpu.VMEM_SHARED((NPAD, HID), jnp.float32),
    ],
)


RB = 1024
GRID = NPAD // RB


def _dinv_of(dp_ref):
    deg = dp_ref[0, :] + dp_ref[1, :] + 1.0
    return lax.rsqrt(deg)


def _s1_body(x_ref, w_ref, dp_ref, o_ref):
    dinv = _dinv_of(dp_ref)
    h = jnp.dot(x_ref[...], w_ref[...], preferred_element_type=jnp.float32)
    o_ref[...] = h * dinv[:, None]


def _s2_body(p_ref, h1s_ref, dp_ref, b1_ref, w2_ref, o_ref):
    dinv = _dinv_of(dp_ref)
    m = p_ref[0] + p_ref[1] + h1s_ref[...]
    a = jnp.maximum(m * dinv[:, None] + b1_ref[...], 0.0)
    h2 = jnp.dot(a, w2_ref[...], preferred_element_type=jnp.float32)
    o_ref[...] = h2 * dinv[:, None]


def _s3_body(p_ref, h2s_ref, dp_ref, b2_ref, o_ref):
    dinv = _dinv_of(dp_ref)
    z = (p_ref[0] + p_ref[1] + h2s_ref[...]) * dinv[:, None] + b2_ref[...]
    col = lax.broadcasted_iota(jnp.int32, (RB, HID), 1)
    mask = col < OUT_DIM
    zm = jnp.where(mask, z, jnp.float32(-1e30))
    mx = jnp.max(zm, axis=1, keepdims=True)
    e = jnp.where(mask, jnp.exp(z - mx), 0.0)
    lse = jnp.log(jnp.sum(e, axis=1, keepdims=True)) + mx
    o_ref[...] = z - lse


def kernel(x, edge_index, W1, b1, W2, b2):
    src = edge_index[0].astype(jnp.int32)
    dst = edge_index[1].astype(jnp.int32)
    x_pad = jnp.pad(x, ((0, NPAD - N), (0, 0)))
    W2p = jnp.pad(W2, ((0, 0), (0, HID - OUT_DIM)))
    b1r = jnp.reshape(b1, (1, HID))
    b2p = jnp.reshape(jnp.pad(b2, (0, HID - OUT_DIM)), (1, HID))

    deg_p = _deg_call(dst).reshape(2, NPAD)

    h1s = pl.pallas_call(
        _s1_body,
        grid=(GRID,),
        in_specs=[
            pl.BlockSpec((RB, IN_DIM), lambda i: (i, 0)),
            pl.BlockSpec((IN_DIM, HID), lambda i: (0, 0)),
            pl.BlockSpec((2, RB), lambda i: (0, i)),
        ],
        out_specs=pl.BlockSpec((RB, HID), lambda i: (i, 0)),
        out_shape=jax.ShapeDtypeStruct((NPAD, HID), jnp.float32),
    )(x_pad, W1, deg_p)

    p1 = _msg_call(h1s, src, dst).reshape(2, NPAD, HID)

    h2s = pl.pallas_call(
        _s2_body,
        grid=(GRID,),
        in_specs=[
            pl.BlockSpec((2, RB, HID), lambda i: (0, i, 0)),
            pl.BlockSpec((RB, HID), lambda i: (i, 0)),
            pl.BlockSpec((2, RB), lambda i: (0, i)),
            pl.BlockSpec((1, HID), lambda i: (0, 0)),
            pl.BlockSpec((HID, HID), lambda i: (0, 0)),
        ],
        out_specs=pl.BlockSpec((RB, HID), lambda i: (i, 0)),
        out_shape=jax.ShapeDtypeStruct((NPAD, HID), jnp.float32),
    )(p1, h1s, deg_p, b1r, W2p)

    p2 = _msg_call(h2s, src, dst).reshape(2, NPAD, HID)

    out = pl.pallas_call(
        _s3_body,
        grid=(GRID,),
        in_specs=[
            pl.BlockSpec((2, RB, HID), lambda i: (0, i, 0)),
            pl.BlockSpec((RB, HID), lambda i: (i, 0)),
            pl.BlockSpec((2, RB), lambda i: (0, i)),
            pl.BlockSpec((1, HID), lambda i: (0, 0)),
        ],
        out_specs=pl.BlockSpec((RB, HID), lambda i: (i, 0)),
        out_shape=jax.ShapeDtypeStruct((NPAD, HID), jnp.float32),
    )(p2, h2s, deg_p, b2p)

    return out[:N, :OUT_DIM]


# speedup vs baseline: 56.9754x; 56.9754x over previous
import jax
import jax.numpy as jnp
from jax import lax
from jax.experimental import pallas as pl
from jax.experimental.pallas import tpu as pltpu
from jax.experimental.pallas import tpu_sc as plsc

N = 10000
E = 320000
IN_DIM = 128
HID = 16
OUT_DIM = 7
NPAD = 10240
NC = 2
NS = 16
NW = NC * NS
EPT = E // NW
CHUNK = 2000
NCH = EPT // CHUNK
RPT = NPAD // NS

_mesh = plsc.VectorSubcoreMesh(
    core_axis_name="c", subcore_axis_name="s", num_cores=NC, num_subcores=NS
)
_sc_params = pltpu.CompilerParams(use_tc_tiling_on_sc=False)


def _deg_body(dst_hbm, out_hbm, idx_v, ones_v, stg, deg_sh):
    c = lax.axis_index("c")
    s = lax.axis_index("s")
    wid = c * NS + s

    def fill(i, _):
        ones_v[pl.ds(i * 16, 16)] = jnp.full((16,), 1.0, jnp.float32)
        stg[pl.ds(i * 16, 16)] = jnp.zeros((16,), jnp.float32)
        return 0

    lax.fori_loop(0, CHUNK // 16, fill, 0)
    pltpu.sync_copy(stg.at[pl.ds(0, RPT)], deg_sh.at[pl.ds(s * RPT, RPT)])
    plsc.subcore_barrier()

    def body(j, _):
        base = wid * EPT + j * CHUNK
        pltpu.sync_copy(dst_hbm.at[pl.ds(base, CHUNK)], idx_v)
        pltpu.sync_copy(ones_v, deg_sh.at[idx_v], add=True)
        return 0

    lax.fori_loop(0, NCH, body, 0)
    plsc.subcore_barrier()
    pltpu.sync_copy(deg_sh.at[pl.ds(s * RPT, RPT)], stg.at[pl.ds(0, RPT)])
    pltpu.sync_copy(stg.at[pl.ds(0, RPT)], out_hbm.at[pl.ds(c * NPAD + s * RPT, RPT)])


_deg_call = pl.kernel(
    _deg_body,
    out_type=jax.ShapeDtypeStruct((2 * NPAD,), jnp.float32),
    mesh=_mesh,
    scratch_types=[
        pltpu.VMEM((CHUNK,), jnp.int32),
        pltpu.VMEM((CHUNK,), jnp.float32),
        pltpu.VMEM((CHUNK,), jnp.float32),
        pltpu.VMEM_SHARED((NPAD,), jnp.float32),
    ],
    compiler_params=_sc_params,
)


def _msg_body(tab_hbm, src_hbm, dst_hbm, out_hbm, idx_s, idx_d, rows, stg, acc_sh):
    c = lax.axis_index("c")
    s = lax.axis_index("s")
    wid = c * NS + s

    def zfill(i, _):
        stg[i, :] = jnp.zeros((16,), jnp.float32)
        return 0

    lax.fori_loop(0, RPT, zfill, 0)
    pltpu.sync_copy(stg, acc_sh.at[pl.ds(s * RPT, RPT)])
    plsc.subcore_barrier()

    def body(j, _):
        base = wid * EPT + j * CHUNK
        pltpu.sync_copy(src_hbm.at[pl.ds(base, CHUNK)], idx_s)
        pltpu.sync_copy(dst_hbm.at[pl.ds(base, CHUNK)], idx_d)
        pltpu.sync_copy(tab_hbm.at[idx_s], rows)
        pltpu.sync_copy(rows, acc_sh.at[idx_d], add=True)
        return 0

    lax.fori_loop(0, NCH, body, 0)
    plsc.subcore_barrier()
    pltpu.sync_copy(acc_sh.at[pl.ds(s * RPT, RPT)], stg)
    pltpu.sync_copy(stg, out_hbm.at[pl.ds(c * NPAD + s * RPT, RPT)])


_msg_call = pl.kernel(
    _msg_body,
    out_type=jax.ShapeDtypeStruct((2 * NPAD, HID), jnp.float32),
    mesh=_mesh,
    scratch_types=[
        pltpu.VMEM((CHUNK,), jnp.int32),
        pltpu.VMEM((CHUNK,), jnp.int32),
        pltpu.VMEM((CHUNK, HID), jnp.float32),
        pltpu.VMEM((RPT, HID), jnp.float32),
        pltpu.VMEM_SHARED((NPAD, HID), jnp.float32),
    ],
    compiler_params=_sc_params,
)


RB = 1024
GRID = NPAD // RB


def _dinv_of(dp_ref):
    deg = dp_ref[0, :] + dp_ref[1, :] + 1.0
    return lax.rsqrt(deg)


def _s1_body(x_ref, w_ref, dp_ref, o_ref):
    dinv = _dinv_of(dp_ref)
    h = jnp.dot(x_ref[...], w_ref[...], preferred_element_type=jnp.float32)
    o_ref[...] = h * dinv[:, None]


def _s2_body(p_ref, h1s_ref, dp_ref, b1_ref, w2_ref, o_ref):
    dinv = _dinv_of(dp_ref)
    m = p_ref[0] + p_ref[1] + h1s_ref[...]
    a = jnp.maximum(m * dinv[:, None] + b1_ref[...], 0.0)
    h2 = jnp.dot(a, w2_ref[...], preferred_element_type=jnp.float32)
    o_ref[...] = h2 * dinv[:, None]


def _s3_body(p_ref, h2s_ref, dp_ref, b2_ref, o_ref):
    dinv = _dinv_of(dp_ref)
    z = (p_ref[0] + p_ref[1] + h2s_ref[...]) * dinv[:, None] + b2_ref[...]
    col = lax.broadcasted_iota(jnp.int32, (RB, HID), 1)
    mask = col < OUT_DIM
    zm = jnp.where(mask, z, jnp.float32(-1e30))
    mx = jnp.max(zm, axis=1, keepdims=True)
    e = jnp.where(mask, jnp.exp(z - mx), 0.0)
    lse = jnp.log(jnp.sum(e, axis=1, keepdims=True)) + mx
    o_ref[...] = z - lse


def kernel(x, edge_index, W1, b1, W2, b2):
    src = edge_index[0].astype(jnp.int32)
    dst = edge_index[1].astype(jnp.int32)
    x_pad = jnp.pad(x, ((0, NPAD - N), (0, 0)))
    W2p = jnp.pad(W2, ((0, 0), (0, HID - OUT_DIM)))
    b1r = jnp.reshape(b1, (1, HID))
    b2p = jnp.reshape(jnp.pad(b2, (0, HID - OUT_DIM)), (1, HID))

    deg_p = _deg_call(dst).reshape(2, NPAD)

    h1s = pl.pallas_call(
        _s1_body,
        grid=(GRID,),
        in_specs=[
            pl.BlockSpec((RB, IN_DIM), lambda i: (i, 0)),
            pl.BlockSpec((IN_DIM, HID), lambda i: (0, 0)),
            pl.BlockSpec((2, RB), lambda i: (0, i)),
        ],
        out_specs=pl.BlockSpec((RB, HID), lambda i: (i, 0)),
        out_shape=jax.ShapeDtypeStruct((NPAD, HID), jnp.float32),
    )(x_pad, W1, deg_p)

    p1 = _msg_call(h1s, src, dst).reshape(2, NPAD, HID)

    h2s = pl.pallas_call(
        _s2_body,
        grid=(GRID,),
        in_specs=[
            pl.BlockSpec((2, RB, HID), lambda i: (0, i, 0)),
            pl.BlockSpec((RB, HID), lambda i: (i, 0)),
            pl.BlockSpec((2, RB), lambda i: (0, i)),
            pl.BlockSpec((1, HID), lambda i: (0, 0)),
            pl.BlockSpec((HID, HID), lambda i: (0, 0)),
        ],
        out_specs=pl.BlockSpec((RB, HID), lambda i: (i, 0)),
        out_shape=jax.ShapeDtypeStruct((NPAD, HID), jnp.float32),
    )(p1, h1s, deg_p, b1r, W2p)

    p2 = _msg_call(h2s, src, dst).reshape(2, NPAD, HID)

    out = pl.pallas_call(
        _s3_body,
        grid=(GRID,),
        in_specs=[
            pl.BlockSpec((2, RB, HID), lambda i: (0, i, 0)),
            pl.BlockSpec((RB, HID), lambda i: (i, 0)),
            pl.BlockSpec((2, RB), lambda i: (0, i)),
            pl.BlockSpec((1, HID), lambda i: (0, 0)),
        ],
        out_specs=pl.BlockSpec((RB, HID), lambda i: (i, 0)),
        out_shape=jax.ShapeDtypeStruct((NPAD, HID), jnp.float32),
    )(p2, h2s, deg_p, b2p)

    return out[:N, :OUT_DIM]


# speedup vs baseline: 67.8932x; 1.1916x over previous
import jax
import jax.numpy as jnp
from jax import lax
from jax.experimental import pallas as pl
from jax.experimental.pallas import tpu as pltpu
from jax.experimental.pallas import tpu_sc as plsc

N = 10000
E = 320000
IN_DIM = 128
HID = 16
OUT_DIM = 7
NPAD = 10240
NC = 2
NS = 16
NW = NC * NS
EPT = E // NW
CHUNK = 2000
NCH = EPT // CHUNK
RPT = NPAD // NS

_mesh = plsc.VectorSubcoreMesh(
    core_axis_name="c", subcore_axis_name="s", num_cores=NC, num_subcores=NS
)
_sc_params = pltpu.CompilerParams(use_tc_tiling_on_sc=False)


def _deg_body(dst_hbm, out_hbm, idx_v, ones_v, stg, deg_sh):
    c = lax.axis_index("c")
    s = lax.axis_index("s")
    wid = c * NS + s

    def fill(i, _):
        ones_v[pl.ds(i * 16, 16)] = jnp.full((16,), 1.0, jnp.float32)
        stg[pl.ds(i * 16, 16)] = jnp.zeros((16,), jnp.float32)
        return 0

    lax.fori_loop(0, CHUNK // 16, fill, 0)
    pltpu.sync_copy(stg.at[pl.ds(0, RPT)], deg_sh.at[pl.ds(s * RPT, RPT)])
    plsc.subcore_barrier()

    def body(j, _):
        base = wid * EPT + j * CHUNK
        pltpu.sync_copy(dst_hbm.at[pl.ds(base, CHUNK)], idx_v)
        pltpu.sync_copy(ones_v, deg_sh.at[idx_v], add=True)
        return 0

    lax.fori_loop(0, NCH, body, 0)
    plsc.subcore_barrier()
    pltpu.sync_copy(deg_sh.at[pl.ds(s * RPT, RPT)], stg.at[pl.ds(0, RPT)])
    pltpu.sync_copy(stg.at[pl.ds(0, RPT)], out_hbm.at[pl.ds(c * NPAD + s * RPT, RPT)])


_deg_call = pl.kernel(
    _deg_body,
    out_type=jax.ShapeDtypeStruct((2 * NPAD,), jnp.float32),
    mesh=_mesh,
    scratch_types=[
        pltpu.VMEM((CHUNK,), jnp.int32),
        pltpu.VMEM((CHUNK,), jnp.float32),
        pltpu.VMEM((CHUNK,), jnp.float32),
        pltpu.VMEM_SHARED((NPAD,), jnp.float32),
    ],
    compiler_params=_sc_params,
)


def _msg_body(tab_hbm, src_hbm, dst_hbm, out_hbm, idx_s, idx_d, rows, stg, acc_sh):
    c = lax.axis_index("c")
    s = lax.axis_index("s")
    wid = c * NS + s

    def zfill(i, _):
        stg[i, :] = jnp.zeros((16,), jnp.float32)
        return 0

    lax.fori_loop(0, RPT, zfill, 0)
    pltpu.sync_copy(stg, acc_sh.at[pl.ds(s * RPT, RPT)])
    plsc.subcore_barrier()

    def body(j, _):
        base = wid * EPT + j * CHUNK
        pltpu.sync_copy(src_hbm.at[pl.ds(base, CHUNK)], idx_s)
        pltpu.sync_copy(dst_hbm.at[pl.ds(base, CHUNK)], idx_d)
        pltpu.sync_copy(tab_hbm.at[idx_s], rows)
        pltpu.sync_copy(rows, acc_sh.at[idx_d], add=True)
        return 0

    lax.fori_loop(0, NCH, body, 0)
    plsc.subcore_barrier()
    pltpu.sync_copy(acc_sh.at[pl.ds(s * RPT, RPT)], stg)
    pltpu.sync_copy(stg, out_hbm.at[pl.ds(c * NPAD + s * RPT, RPT)])


_msg_call = pl.kernel(
    _msg_body,
    out_type=jax.ShapeDtypeStruct((2 * NPAD, HID), jnp.float32),
    mesh=_mesh,
    scratch_types=[
        pltpu.VMEM((CHUNK,), jnp.int32),
        pltpu.VMEM((CHUNK,), jnp.int32),
        pltpu.VMEM((CHUNK, HID), jnp.float32),
        pltpu.VMEM((RPT, HID), jnp.float32),
        pltpu.VMEM_SHARED((NPAD, HID), jnp.float32),
    ],
    compiler_params=_sc_params,
)


RB = 1024
GRID = NPAD // RB
RBP = RB * HID // 128
DBP = RB // 128


def _dscale_blk(dp_ref):
    deg = dp_ref[0] + dp_ref[1] + 1.0
    dinv = lax.rsqrt(deg)
    r_col = lax.broadcasted_iota(jnp.int32, (RBP, DBP), 0)
    q_row = lax.broadcasted_iota(jnp.int32, (RBP, DBP), 1)
    qoh = jnp.where(r_col // 16 == q_row, 1.0, 0.0)
    u = jnp.dot(qoh, dinv, preferred_element_type=jnp.float32)
    r2 = lax.broadcasted_iota(jnp.int32, (RBP, 128), 0)
    l2 = lax.broadcasted_iota(jnp.int32, (RBP, 128), 1)
    cols = []
    for a in range(8):
        loh = jnp.where(l2 == 8 * (r2 % 16) + a, 1.0, 0.0)
        cols.append(jnp.sum(u * loh, axis=1, keepdims=True))
    dvec8 = jnp.concatenate(cols, axis=1)
    a_row = lax.broadcasted_iota(jnp.int32, (8, 128), 0)
    m_col = lax.broadcasted_iota(jnp.int32, (8, 128), 1)
    bcast = jnp.where(m_col // 16 == a_row, 1.0, 0.0)
    return jnp.dot(dvec8, bcast, preferred_element_type=jnp.float32)


def _lane_mask(val=1.0):
    m = lax.broadcasted_iota(jnp.int32, (RBP, 128), 1)
    return jnp.where(m % HID < OUT_DIM, val, 0.0)


def _s1_body(x3_ref, w1a_ref, dp_ref, o_ref):
    dscale = _dscale_blk(dp_ref)
    x3 = x3_ref[...]
    acc = jnp.zeros((RBP, 128), jnp.float32)
    for a in range(8):
        acc = acc + jnp.dot(x3[:, a, :], w1a_ref[a],
                            preferred_element_type=jnp.float32)
    o_ref[...] = acc * dscale


def _s2_body(p_ref, h1s_ref, dp_ref, b1_ref, w2bd_ref, o_ref):
    dscale = _dscale_blk(dp_ref)
    m = p_ref[0] + p_ref[1] + h1s_ref[...]
    a = jnp.maximum(m * dscale + b1_ref[...], 0.0)
    h2 = jnp.dot(a, w2bd_ref[...], preferred_element_type=jnp.float32)
    o_ref[...] = h2 * dscale


def _s3_body(p_ref, h2s_ref, dp_ref, b2_ref, gs_ref, o_ref):
    dscale = _dscale_blk(dp_ref)
    z = (p_ref[0] + p_ref[1] + h2s_ref[...]) * dscale + b2_ref[...]
    mask = _lane_mask()
    zm = jnp.where(mask > 0.0, z, jnp.float32(-1e30))
    mx = jnp.max(zm, axis=1, keepdims=True)
    e = jnp.where(mask > 0.0, jnp.exp(z - mx), 0.0)
    gs = jnp.dot(e, gs_ref[...], preferred_element_type=jnp.float32)
    lse = jnp.log(gs) + mx
    o_ref[...] = z - lse


def kernel(x, edge_index, W1, b1, W2, b2):
    src = edge_index[0].astype(jnp.int32)
    dst = edge_index[1].astype(jnp.int32)
    W2p = jnp.pad(W2, ((0, 0), (0, HID - OUT_DIM)))
    W1a = jnp.stack([jnp.pad(W1, ((0, 0), (HID * a, 128 - HID * a - HID)))
                     for a in range(8)])
    aeq = (jnp.arange(128)[:, None] // HID) == (jnp.arange(128)[None, :] // HID)
    W2bd = jnp.where(aeq, jnp.tile(W2p, (8, 8)), 0.0)
    gs_mat = aeq.astype(jnp.float32)
    b1r = jnp.reshape(jnp.tile(b1, 8), (1, 128))
    b2p = jnp.reshape(jnp.tile(jnp.pad(b2, (0, HID - OUT_DIM)), 8), (1, 128))
    x3 = jnp.reshape(x, (N // 8, 8, IN_DIM))

    deg_pk = _deg_call(dst).reshape(2, NPAD // 128, 128)

    h1s_pk = pl.pallas_call(
        _s1_body,
        grid=(GRID,),
        in_specs=[
            pl.BlockSpec((RBP, 8, IN_DIM), lambda i: (i, 0, 0)),
            pl.BlockSpec((8, IN_DIM, 128), lambda i: (0, 0, 0)),
            pl.BlockSpec((2, DBP, 128), lambda i: (0, i, 0)),
        ],
        out_specs=pl.BlockSpec((RBP, 128), lambda i: (i, 0)),
        out_shape=jax.ShapeDtypeStruct((NPAD * HID // 128, 128), jnp.float32),
    )(x3, W1a, deg_pk)

    p1_pk = _msg_call(h1s_pk.reshape(NPAD, HID), src, dst).reshape(
        2, NPAD * HID // 128, 128)

    h2s_pk = pl.pallas_call(
        _s2_body,
        grid=(GRID,),
        in_specs=[
            pl.BlockSpec((2, RBP, 128), lambda i: (0, i, 0)),
            pl.BlockSpec((RBP, 128), lambda i: (i, 0)),
            pl.BlockSpec((2, DBP, 128), lambda i: (0, i, 0)),
            pl.BlockSpec((1, 128), lambda i: (0, 0)),
            pl.BlockSpec((128, 128), lambda i: (0, 0)),
        ],
        out_specs=pl.BlockSpec((RBP, 128), lambda i: (i, 0)),
        out_shape=jax.ShapeDtypeStruct((NPAD * HID // 128, 128), jnp.float32),
    )(p1_pk, h1s_pk, deg_pk, b1r, W2bd)

    p2_pk = _msg_call(h2s_pk.reshape(NPAD, HID), src, dst).reshape(
        2, NPAD * HID // 128, 128)

    out_pk = pl.pallas_call(
        _s3_body,
        grid=(GRID,),
        in_specs=[
            pl.BlockSpec((2, RBP, 128), lambda i: (0, i, 0)),
            pl.BlockSpec((RBP, 128), lambda i: (i, 0)),
            pl.BlockSpec((2, DBP, 128), lambda i: (0, i, 0)),
            pl.BlockSpec((1, 128), lambda i: (0, 0)),
            pl.BlockSpec((128, 128), lambda i: (0, 0)),
        ],
        out_specs=pl.BlockSpec((RBP, 128), lambda i: (i, 0)),
        out_shape=jax.ShapeDtypeStruct((NPAD * HID // 128, 128), jnp.float32),
    )(p2_pk, h2s_pk, deg_pk, b2p, gs_mat)

    return out_pk.reshape(NPAD, HID)[:N, :OUT_DIM]


# speedup vs baseline: 95.4608x; 1.4060x over previous
import jax
import jax.numpy as jnp
from jax import lax
from jax.experimental import pallas as pl
from jax.experimental.pallas import tpu as pltpu
from jax.experimental.pallas import tpu_sc as plsc

N = 10000
E = 320000
IN_DIM = 128
HID = 16
OUT_DIM = 7
NPAD = 10240
NC = 2
NS = 16
NW = NC * NS
EPT = E // NW
CHUNK = 2000
NCH = EPT // CHUNK
RPT = NPAD // NS

_mesh = plsc.VectorSubcoreMesh(
    core_axis_name="c", subcore_axis_name="s", num_cores=NC, num_subcores=NS
)
_sc_params = pltpu.CompilerParams(use_tc_tiling_on_sc=False)


def _deg_body(ei_hbm, out_hbm, idx2, ones_v, stg, deg_sh, sem):
    c = lax.axis_index("c")
    s = lax.axis_index("s")
    wid = c * NS + s

    def fill(i, _):
        ones_v[pl.ds(i * 16, 16)] = jnp.full((16,), 1.0, jnp.float32)
        stg[pl.ds(i * 16, 16)] = jnp.zeros((16,), jnp.float32)
        return 0

    lax.fori_loop(0, CHUNK // 16, fill, 0)
    descs = [
        pltpu.async_copy(
            ei_hbm.at[1, pl.ds(wid * EPT + j * CHUNK, CHUNK)], idx2.at[j], sem)
        for j in range(NCH)
    ]
    pltpu.sync_copy(stg.at[pl.ds(0, RPT)], deg_sh.at[pl.ds(s * RPT, RPT)])
    for d in descs:
        d.wait()
    plsc.subcore_barrier()
    for j in range(NCH):
        pltpu.sync_copy(ones_v, deg_sh.at[idx2.at[j]], add=True)
    plsc.subcore_barrier()
    pltpu.sync_copy(deg_sh.at[pl.ds(s * RPT, RPT)], stg.at[pl.ds(0, RPT)])
    pltpu.sync_copy(stg.at[pl.ds(0, RPT)], out_hbm.at[pl.ds(c * NPAD + s * RPT, RPT)])


_deg_call = pl.kernel(
    _deg_body,
    out_type=jax.ShapeDtypeStruct((2 * NPAD,), jnp.float32),
    mesh=_mesh,
    scratch_types=[
        pltpu.VMEM((NCH, CHUNK), jnp.int32),
        pltpu.VMEM((CHUNK,), jnp.float32),
        pltpu.VMEM((CHUNK,), jnp.float32),
        pltpu.VMEM_SHARED((NPAD,), jnp.float32),
        pltpu.SemaphoreType.DMA,
    ],
    compiler_params=_sc_params,
)


def _msg_body(tab_hbm, ei_hbm, out_hbm, idx_s2, idx_d2, rows0, rows1, stg,
              acc_sh, semi, sg0, sg1, ssc):
    c = lax.axis_index("c")
    s = lax.axis_index("s")
    wid = c * NS + s

    def zfill(i, _):
        stg[i, :] = jnp.zeros((16,), jnp.float32)
        return 0

    lax.fori_loop(0, RPT, zfill, 0)
    descs = []
    for j in range(NCH):
        base = wid * EPT + j * CHUNK
        descs.append(pltpu.async_copy(
            ei_hbm.at[0, pl.ds(base, CHUNK)], idx_s2.at[j], semi))
        descs.append(pltpu.async_copy(
            ei_hbm.at[1, pl.ds(base, CHUNK)], idx_d2.at[j], semi))
    pltpu.sync_copy(stg, acc_sh.at[pl.ds(s * RPT, RPT)])
    for d in descs:
        d.wait()
    plsc.subcore_barrier()

    bufs = (rows0, rows1)
    sems = (sg0, sg1)
    g = [None] * NCH
    sc = [None] * NCH
    g[0] = pltpu.async_copy(tab_hbm.at[idx_s2.at[0]], bufs[0], sems[0])
    for j in range(NCH):
        g[j].wait()
        if j >= 1:
            sc[j - 1].wait()
        if j + 1 < NCH:
            g[j + 1] = pltpu.async_copy(
                tab_hbm.at[idx_s2.at[j + 1]], bufs[(j + 1) % 2], sems[(j + 1) % 2])
        sc[j] = pltpu.async_copy(
            bufs[j % 2], acc_sh.at[idx_d2.at[j]], ssc, add=True)
    sc[NCH - 1].wait()
    plsc.subcore_barrier()
    pltpu.sync_copy(acc_sh.at[pl.ds(s * RPT, RPT)], stg)
    pltpu.sync_copy(stg, out_hbm.at[pl.ds(c * NPAD + s * RPT, RPT)])


_msg_call = pl.kernel(
    _msg_body,
    out_type=jax.ShapeDtypeStruct((2 * NPAD, HID), jnp.float32),
    mesh=_mesh,
    scratch_types=[
        pltpu.VMEM((NCH, CHUNK), jnp.int32),
        pltpu.VMEM((NCH, CHUNK), jnp.int32),
        pltpu.VMEM((CHUNK, HID), jnp.float32),
        pltpu.VMEM((CHUNK, HID), jnp.float32),
        pltpu.VMEM((RPT, HID), jnp.float32),
        pltpu.VMEM_SHARED((NPAD, HID), jnp.float32),
        pltpu.SemaphoreType.DMA,
        pltpu.SemaphoreType.DMA,
        pltpu.SemaphoreType.DMA,
        pltpu.SemaphoreType.DMA,
    ],
    compiler_params=_sc_params,
)


GRID = 2
RB = NPAD // GRID
RBP = RB * HID // 128
DBP = RB // 128


def _dscale_blk(dp_ref):
    deg = dp_ref[0] + dp_ref[1] + 1.0
    dinv = lax.rsqrt(deg)
    r_col = lax.broadcasted_iota(jnp.int32, (RBP, DBP), 0)
    q_row = lax.broadcasted_iota(jnp.int32, (RBP, DBP), 1)
    qoh = jnp.where(r_col // 16 == q_row, 1.0, 0.0)
    u = jnp.dot(qoh, dinv, preferred_element_type=jnp.float32)
    r2 = lax.broadcasted_iota(jnp.int32, (RBP, 128), 0)
    l2 = lax.broadcasted_iota(jnp.int32, (RBP, 128), 1)
    cols = []
    for a in range(8):
        loh = jnp.where(l2 == 8 * (r2 % 16) + a, 1.0, 0.0)
        cols.append(jnp.sum(u * loh, axis=1, keepdims=True))
    dvec8 = jnp.concatenate(cols, axis=1)
    a_row = lax.broadcasted_iota(jnp.int32, (8, 128), 0)
    m_col = lax.broadcasted_iota(jnp.int32, (8, 128), 1)
    bcast = jnp.where(m_col // 16 == a_row, 1.0, 0.0)
    return jnp.dot(dvec8, bcast, preferred_element_type=jnp.float32)


def _s1_body(x3_ref, w1a_ref, dp_ref, o_ref):
    dscale = _dscale_blk(dp_ref)
    x3 = x3_ref[...]
    acc = jnp.zeros((RBP, 128), jnp.float32)
    for a in range(8):
        acc = acc + jnp.dot(x3[:, a, :], w1a_ref[a],
                            preferred_element_type=jnp.float32)
    o_ref[...] = acc * dscale


def _s2_body(p_ref, h1s_ref, dp_ref, b1_ref, w2bd_ref, o_ref):
    dscale = _dscale_blk(dp_ref)
    m = p_ref[0] + p_ref[1] + h1s_ref[...]
    a = jnp.maximum(m * dscale + b1_ref[...], 0.0)
    h2 = jnp.dot(a, w2bd_ref[...], preferred_element_type=jnp.float32)
    o_ref[...] = h2 * dscale


def _s3_body(p_ref, h2s_ref, dp_ref, b2_ref, gs_ref, o_ref):
    dscale = _dscale_blk(dp_ref)
    z = (p_ref[0] + p_ref[1] + h2s_ref[...]) * dscale + b2_ref[...]
    mcol = lax.broadcasted_iota(jnp.int32, (RBP, 128), 1)
    mask = mcol % HID < OUT_DIM
    zm = jnp.where(mask, z, jnp.float32(-1e30))
    mx = jnp.max(zm, axis=1, keepdims=True)
    e = jnp.where(mask, jnp.exp(z - mx), 0.0)
    gs = jnp.dot(e, gs_ref[...], preferred_element_type=jnp.float32)
    lse = jnp.log(gs) + mx
    res = z - lse
    parts = [res[:, HID * a:HID * a + OUT_DIM] for a in range(8)]
    o_ref[...] = jnp.stack(parts, axis=1)


def kernel(x, edge_index, W1, b1, W2, b2):
    ei = edge_index.astype(jnp.int32)
    W2p = jnp.pad(W2, ((0, 0), (0, HID - OUT_DIM)))
    W1a = jnp.stack([jnp.pad(W1, ((0, 0), (HID * a, 128 - HID * a - HID)))
                     for a in range(8)])
    aeq = (jnp.arange(128)[:, None] // HID) == (jnp.arange(128)[None, :] // HID)
    W2bd = jnp.where(aeq, jnp.tile(W2p, (8, 8)), 0.0)
    gs_mat = aeq.astype(jnp.float32)
    b1r = jnp.reshape(jnp.tile(b1, 8), (1, 128))
    b2p = jnp.reshape(jnp.tile(jnp.pad(b2, (0, HID - OUT_DIM)), 8), (1, 128))
    x3 = jnp.reshape(x, (N // 8, 8, IN_DIM))

    deg_pk = _deg_call(ei).reshape(2, NPAD // 128, 128)

    h1s_pk = pl.pallas_call(
        _s1_body,
        grid=(GRID,),
        in_specs=[
            pl.BlockSpec((RBP, 8, IN_DIM), lambda i: (i, 0, 0)),
            pl.BlockSpec((8, IN_DIM, 128), lambda i: (0, 0, 0)),
            pl.BlockSpec((2, DBP, 128), lambda i: (0, i, 0)),
        ],
        out_specs=pl.BlockSpec((RBP, 128), lambda i: (i, 0)),
        out_shape=jax.ShapeDtypeStruct((NPAD * HID // 128, 128), jnp.float32),
    )(x3, W1a, deg_pk)

    p1_pk = _msg_call(h1s_pk.reshape(NPAD, HID), ei).reshape(
        2, NPAD * HID // 128, 128)

    h2s_pk = pl.pallas_call(
        _s2_body,
        grid=(GRID,),
        in_specs=[
            pl.BlockSpec((2, RBP, 128), lambda i: (0, i, 0)),
            pl.BlockSpec((RBP, 128), lambda i: (i, 0)),
            pl.BlockSpec((2, DBP, 128), lambda i: (0, i, 0)),
            pl.BlockSpec((1, 128), lambda i: (0, 0)),
            pl.BlockSpec((128, 128), lambda i: (0, 0)),
        ],
        out_specs=pl.BlockSpec((RBP, 128), lambda i: (i, 0)),
        out_shape=jax.ShapeDtypeStruct((NPAD * HID // 128, 128), jnp.float32),
    )(p1_pk, h1s_pk, deg_pk, b1r, W2bd)

    p2_pk = _msg_call(h2s_pk.reshape(NPAD, HID), ei).reshape(
        2, NPAD * HID // 128, 128)

    out38 = pl.pallas_call(
        _s3_body,
        grid=(GRID,),
        in_specs=[
            pl.BlockSpec((2, RBP, 128), lambda i: (0, i, 0)),
            pl.BlockSpec((RBP, 128), lambda i: (i, 0)),
            pl.BlockSpec((2, DBP, 128), lambda i: (0, i, 0)),
            pl.BlockSpec((1, 128), lambda i: (0, 0)),
            pl.BlockSpec((128, 128), lambda i: (0, 0)),
        ],
        out_specs=pl.BlockSpec((RBP, 8, OUT_DIM), lambda i: (i, 0, 0)),
        out_shape=jax.ShapeDtypeStruct((N // 8, 8, OUT_DIM), jnp.float32),
    )(p2_pk, h2s_pk, deg_pk, b2p, gs_mat)

    return out38.reshape(N, OUT_DIM)
